# Initial kernel scaffold; baseline (speedup 1.0000x reference)
import jax
import jax.numpy as jnp
from jax.experimental import pallas as pl


def kernel(z, W_in_a, b_in_a, W_in_b, b_in_b, W_out_a, b_out_a, W_out_b, b_out_b, codebook_a, codebook_b):
    raise NotImplementedError("write your pallas kernel here")


# speedup vs baseline: 1.0998x; 1.0998x over previous
import jax
import jax.numpy as jnp
from jax.experimental import pallas as pl
from jax.experimental.pallas import tpu as pltpu
from jax.experimental.pallas import tpu_sc as plsc

B, D, T = 8, 1024, 512
CD, K = 64, 8192
KB = 2048
NKB = K // KB
NIDX = 2 * B * T
GWIN = 128
CDP = 128


def _search_body(z_ref, w_ref, bias_ref, cba_ref, cbb_ref,
                 lat_ref, gidx_ref, idx_ref, cbn_ref):
    b = pl.program_id(0)

    @pl.when(b == 0)
    def _():
        for i, cb_ref in enumerate((cba_ref, cbb_ref)):
            cb = cb_ref[...]
            n = jnp.sqrt(jnp.sum(cb * cb, axis=1, keepdims=True))
            cbn_ref[i] = cb / jnp.maximum(n, 1e-12)

    z = z_ref[0]
    lat = jax.lax.dot_general(
        w_ref[...], z, (((1,), (0,)), ((), ())),
        preferred_element_type=jnp.float32) + bias_ref[:, 0:1]
    lat_ref[0] = lat

    idx2 = []
    for i in range(2):
        enc = lat[i * CD:(i + 1) * CD, :]
        n = jnp.sqrt(jnp.sum(enc * enc, axis=0, keepdims=True))
        encn = enc / jnp.maximum(n, 1e-12)
        rn2 = jnp.sum(encn * encn, axis=0, keepdims=True)
        run_v = jnp.full((1, T), -jnp.inf, jnp.float32)
        run_i = jnp.zeros((1, T), jnp.int32)
        for k in range(NKB):
            cbn = cbn_ref[i, k * KB:(k + 1) * KB, :]
            s = jax.lax.dot_general(
                cbn, encn, (((1,), (0,)), ((), ())),
                preferred_element_type=jnp.float32)
            cn2 = jnp.sum(cbn * cbn, axis=1, keepdims=True)
            negd = -((rn2 - 2.0 * s) + cn2)
            bm = jnp.max(negd, axis=0, keepdims=True)
            rows = jax.lax.broadcasted_iota(jnp.int32, (KB, T), 0)
            bi = jnp.min(jnp.where(negd == bm, rows, K),
                         axis=0, keepdims=True) + (k * KB)
            upd = bm > run_v
            run_i = jnp.where(upd, bi, run_i)
            run_v = jnp.where(upd, bm, run_v)
        idx2.append(run_i)

    gidx_ref[0, 0:1, :] = idx2[0]
    gidx_ref[0, 1:2, :] = idx2[1] + K
    idx_ref[0] = idx2[0] * K + idx2[1]


def _out_body(lat_ref, qa_ref, qb_ref, woa_ref, wob_ref, boa_ref, bob_ref,
              zq_ref, loss_ref):
    lat = lat_ref[0]
    loss = jnp.zeros((), jnp.float32)
    for i, (q_ref, w_ref, bo_ref) in enumerate(
            ((qa_ref, woa_ref, boa_ref), (qb_ref, wob_ref, bob_ref))):
        z_i = lat[i * CD:(i + 1) * CD, :]
        qT = jnp.transpose(q_ref[0][:, 0:CD], (1, 0))
        diff = z_i - qT
        loss = loss + jnp.sum(diff * diff) / float(CD * T)
        st = z_i + (qT - z_i)
        zq = jax.lax.dot_general(
            w_ref[...], st, (((1,), (0,)), ((), ())),
            preferred_element_type=jnp.float32) + bo_ref[:, 0:1]
        zq_ref[0, i * (D // 2):(i + 1) * (D // 2), :] = zq
    loss_ref[0, 0, :] = jnp.full((128,), loss, jnp.float32)


def _sc_gather(cb_all, gflat):
    mesh = plsc.VectorSubcoreMesh(core_axis_name="c", subcore_axis_name="s")

    @pl.kernel(out_type=jax.ShapeDtypeStruct((NIDX, CDP), jnp.float32),
               mesh=mesh)
    def gk(cb_hbm, i_hbm, o_hbm):
        def body(i_vmem, o_vmem):
            pltpu.sync_copy(cb_hbm.at[i_vmem.at[0]], o_vmem)

        pltpu.emit_pipeline(
            body,
            grid=(NIDX // GWIN,),
            in_specs=[pl.BlockSpec((1, GWIN), lambda i: (0, i))],
            out_specs=[pl.BlockSpec((GWIN, CDP), lambda i: (i, 0))],
            core_axis_name=("c", "s"),
            dimension_semantics=(pltpu.PARALLEL,),
        )(i_hbm, o_hbm)

    return gk(cb_all, gflat)


def kernel(z, W_in_a, b_in_a, W_in_b, b_in_b,
           W_out_a, b_out_a, W_out_b, b_out_b,
           codebook_a, codebook_b):
    f32 = jnp.float32
    w_stack = jnp.concatenate([W_in_a, W_in_b], axis=0)
    bias2d = jnp.broadcast_to(
        jnp.concatenate([b_in_a, b_in_b])[:, None], (2 * CD, 128))

    lat, gidx, idx3 = pl.pallas_call(
        _search_body,
        grid=(B,),
        in_specs=[
            pl.BlockSpec((1, D, T), lambda b: (b, 0, 0)),
            pl.BlockSpec((2 * CD, D), lambda b: (0, 0)),
            pl.BlockSpec((2 * CD, 128), lambda b: (0, 0)),
            pl.BlockSpec((K, CD), lambda b: (0, 0)),
            pl.BlockSpec((K, CD), lambda b: (0, 0)),
        ],
        out_specs=[
            pl.BlockSpec((1, 2 * CD, T), lambda b: (b, 0, 0)),
            pl.BlockSpec((1, 2, T), lambda b: (b, 0, 0)),
            pl.BlockSpec((1, 1, T), lambda b: (b, 0, 0)),
        ],
        out_shape=[
            jax.ShapeDtypeStruct((B, 2 * CD, T), f32),
            jax.ShapeDtypeStruct((B, 2, T), jnp.int32),
            jax.ShapeDtypeStruct((B, 1, T), jnp.int32),
        ],
        scratch_shapes=[pltpu.VMEM((2, K, CD), f32)],
    )(z, w_stack, bias2d, codebook_a, codebook_b)

    indices = idx3[:, 0, :]
    gflat = gidx.transpose(1, 0, 2).reshape(1, NIDX)
    cb_all = jnp.pad(jnp.concatenate([codebook_a, codebook_b], axis=0),
                     ((0, 0), (0, CDP - CD)))

    rows = _sc_gather(cb_all, gflat).reshape(2, B, T, CDP)

    bo_a = jnp.broadcast_to(b_out_a[:, None], (D // 2, 128))
    bo_b = jnp.broadcast_to(b_out_b[:, None], (D // 2, 128))

    zq, loss3 = pl.pallas_call(
        _out_body,
        grid=(B,),
        in_specs=[
            pl.BlockSpec((1, 2 * CD, T), lambda b: (b, 0, 0)),
            pl.BlockSpec((1, T, CDP), lambda b: (b, 0, 0)),
            pl.BlockSpec((1, T, CDP), lambda b: (b, 0, 0)),
            pl.BlockSpec((D // 2, CD), lambda b: (0, 0)),
            pl.BlockSpec((D // 2, CD), lambda b: (0, 0)),
            pl.BlockSpec((D // 2, 128), lambda b: (0, 0)),
            pl.BlockSpec((D // 2, 128), lambda b: (0, 0)),
        ],
        out_specs=[
            pl.BlockSpec((1, D, T), lambda b: (b, 0, 0)),
            pl.BlockSpec((1, 1, 128), lambda b: (b, 0, 0)),
        ],
        out_shape=[
            jax.ShapeDtypeStruct((B, D, T), f32),
            jax.ShapeDtypeStruct((B, 1, 128), f32),
        ],
    )(lat, rows[0], rows[1], W_out_a, W_out_b, bo_a, bo_b)

    loss = loss3[:, 0, 0]
    return (zq, loss, loss, indices, lat)
